# baseline (speedup 1.0000x reference)
import functools

import jax
import jax.numpy as jnp
from jax import lax
from jax.experimental import pallas as pl
from jax.experimental.pallas import tpu as pltpu
from jax.experimental.pallas import tpu_sc as plsc

_NC = 2
_NS = 16
_NW = _NC * _NS
_CH = 128


def _proj_body(f_ref, w_ref, b_ref, o_ref):
    o_ref[...] = (
        jnp.dot(f_ref[...], w_ref[...], preferred_element_type=jnp.float32)
        + b_ref[...]
    ).astype(jnp.bfloat16)


def _make_gather(ep, dp, dx, cpt):
    mesh = plsc.VectorSubcoreMesh(
        core_axis_name="c", subcore_axis_name="s",
        num_cores=_NC, num_subcores=_NS,
    )

    nbuf = 8
    ngroups = cpt // nbuf

    @functools.partial(
        pl.kernel,
        out_type=(
            jax.ShapeDtypeStruct((ep, dp), jnp.bfloat16),
            jax.ShapeDtypeStruct((ep, dx), jnp.float32),
        ),
        mesh=mesh,
        scratch_types=[
            pltpu.VMEM((cpt, _CH), jnp.int32),
            pltpu.VMEM((2, nbuf, _CH, dp), jnp.bfloat16),
            pltpu.VMEM((2, nbuf, _CH, dx), jnp.float32),
            pltpu.SemaphoreType.DMA,
            pltpu.SemaphoreType.DMA,
        ],
        compiler_params=pltpu.CompilerParams(use_tc_tiling_on_sc=False),
    )
    def gather_k(idx_hbm, proj_hbm, pts_hbm, gp_hbm, gx_hbm,
                 idx_v, bufp, bufx, sem0, sem1):
        wid = lax.axis_index("s") * _NC + lax.axis_index("c")
        row0 = wid * cpt
        pltpu.sync_copy(idx_hbm.at[pl.ds(row0, cpt)], idx_v)
        sems = (sem0, sem1)

        def fire(g, h):
            for b in range(nbuf):
                i = g * nbuf + b
                pltpu.async_copy(proj_hbm.at[idx_v.at[i]], bufp.at[h, b],
                                 sems[h])
                pltpu.async_copy(pts_hbm.at[idx_v.at[i]], bufx.at[h, b],
                                 sems[h])

        def drain_and_store(g, h):
            for b in range(nbuf):
                pltpu.make_async_copy(
                    proj_hbm.at[pl.ds(0, _CH)], bufp.at[h, b], sems[h]).wait()
                pltpu.make_async_copy(
                    pts_hbm.at[pl.ds(0, _CH)], bufx.at[h, b], sems[h]).wait()
            for b in range(nbuf):
                base = (row0 + (g * nbuf + b)) * _CH
                pltpu.sync_copy(bufp.at[h, b], gp_hbm.at[pl.ds(base, _CH)])
                pltpu.sync_copy(bufx.at[h, b], gx_hbm.at[pl.ds(base, _CH)])

        fire(0, 0)

        def body(t, carry):
            g0 = 2 * t
            fire(g0 + 1, 1)
            drain_and_store(g0, 0)

            @pl.when(t < ngroups // 2 - 1)
            def _():
                fire(g0 + 2, 0)

            drain_and_store(g0 + 1, 1)
            return carry

        lax.fori_loop(0, ngroups // 2, body, 0)

    return gather_k


def _make_finish(blk, k_, dp, dx, dh):
    rpb = blk * k_

    def finish_body(gp_ref, gx_ref, pc_ref, w1p_ref, w1d_ref, b1_ref,
                    w2b_ref, o_ref):
        gx = gx_ref[...]
        center = pc_ref[...]
        rep = jnp.broadcast_to(
            center[:, None, :], (blk, k_, dx)).reshape(rpb, dx)
        diff = rep - gx
        ssq = jnp.sum(diff * diff, axis=1, keepdims=True)
        dist = jnp.sqrt(ssq + 1e-12)
        g1 = jnp.dot(diff, w1p_ref[...], preferred_element_type=jnp.float32)
        g1 = g1 + dist * w1d_ref[...] + b1_ref[...]
        g1 = jnp.where(g1 >= 0, g1, 0.2 * g1)
        z = jnp.dot(g1, w2b_ref[...], preferred_element_type=jnp.float32)
        z = z + gp_ref[...].astype(jnp.float32)
        z = jnp.where(z >= 0, z, 0.2 * z)
        o_ref[...] = jnp.mean(z.reshape(blk, k_, dh), axis=1)

    return finish_body


def kernel(points, features, knn_idx, W1, b1, W2, b2):
    b_, n_, _ = points.shape
    k_ = knn_idx.shape[1]
    d_ = features.shape[-1]
    dh = W2.shape[1]
    dx = 16
    e_ = n_ * k_

    pts = points.reshape(n_, 3)
    feats = features.reshape(n_, d_)

    pts_pad = jnp.zeros((n_, dx), jnp.float32).at[:, :3].set(pts)
    w2_top = W2[:d_]
    w2_bot = W2[d_:]
    w1_pad = jnp.zeros((dx, dh), jnp.float32).at[:3].set(W1[:3])
    w1_dist = W1[3:4]
    b1r = b1.reshape(1, dh)
    b2r = b2.reshape(1, dh)

    cpt = (e_ + _NW * _CH - 1) // (_NW * _CH)
    cpt = ((cpt + 7) // 8) * 8
    ep = cpt * _NW * _CH
    idx_flat = jnp.pad(knn_idx.reshape(-1), (0, ep - e_))
    idx2d = idx_flat.reshape(ep // _CH, _CH)

    proj = pl.pallas_call(
        _proj_body,
        out_shape=jax.ShapeDtypeStruct((n_, dh), jnp.bfloat16),
    )(feats, w2_top, b2r)

    gp, gx = _make_gather(ep, dh, dx, cpt)(idx2d, proj, pts_pad)

    blk = 400
    nb = n_ // blk
    rpb = blk * k_
    out = pl.pallas_call(
        _make_finish(blk, k_, dh, dx, dh),
        grid=(nb,),
        in_specs=[
            pl.BlockSpec((rpb, dh), lambda i: (i, 0)),
            pl.BlockSpec((rpb, dx), lambda i: (i, 0)),
            pl.BlockSpec((blk, dx), lambda i: (i, 0)),
            pl.BlockSpec((dx, dh), lambda i: (0, 0)),
            pl.BlockSpec((1, dh), lambda i: (0, 0)),
            pl.BlockSpec((1, dh), lambda i: (0, 0)),
            pl.BlockSpec((dh, dh), lambda i: (0, 0)),
        ],
        out_specs=pl.BlockSpec((blk, dh), lambda i: (i, 0)),
        out_shape=jax.ShapeDtypeStruct((n_, dh), jnp.float32),
    )(gp, gx, pts_pad, w1_pad, w1_dist, b1r, w2_bot)

    return out.reshape(b_, n_, dh)


# speedup vs baseline: 1.0627x; 1.0627x over previous
import functools

import jax
import jax.numpy as jnp
from jax import lax
from jax.experimental import pallas as pl
from jax.experimental.pallas import tpu as pltpu
from jax.experimental.pallas import tpu_sc as plsc

_NC = 2
_NS = 16
_NW = _NC * _NS
_CH = 128
_WT = 64


def _proj_body(f_ref, w_ref, b_ref, o_ref):
    o_ref[...] = (
        jnp.dot(f_ref[...], w_ref[...], preferred_element_type=jnp.float32)
        + b_ref[...]
    ).astype(jnp.bfloat16)


def _make_gather(ep, cpt):
    mesh = plsc.VectorSubcoreMesh(
        core_axis_name="c", subcore_axis_name="s",
        num_cores=_NC, num_subcores=_NS,
    )

    nbuf = 4
    ngroups = cpt // nbuf

    @functools.partial(
        pl.kernel,
        out_type=jax.ShapeDtypeStruct((ep, _WT), jnp.float32),
        mesh=mesh,
        scratch_types=[
            pltpu.VMEM((cpt, _CH), jnp.int32),
            pltpu.VMEM((2, nbuf, _CH, _WT), jnp.float32),
            pltpu.SemaphoreType.DMA,
            pltpu.SemaphoreType.DMA,
        ],
        compiler_params=pltpu.CompilerParams(use_tc_tiling_on_sc=False),
    )
    def gather_k(idx_hbm, tbl_hbm, out_hbm, idx_v, buf, sem0, sem1):
        wid = lax.axis_index("s") * _NC + lax.axis_index("c")
        row0 = wid * cpt
        pltpu.sync_copy(idx_hbm.at[pl.ds(row0, cpt)], idx_v)
        sems = (sem0, sem1)

        def fire(g, h):
            for b in range(nbuf):
                i = g * nbuf + b
                pltpu.async_copy(tbl_hbm.at[idx_v.at[i]], buf.at[h, b],
                                 sems[h])

        def drain_and_store(g, h):
            for b in range(nbuf):
                pltpu.make_async_copy(
                    tbl_hbm.at[pl.ds(0, _CH)], buf.at[h, b], sems[h]).wait()
            for b in range(nbuf):
                c = row0 + (g * nbuf + b)
                pltpu.sync_copy(buf.at[h, b],
                                out_hbm.at[pl.ds(c * _CH, _CH)])

        fire(0, 0)

        def body(t, carry):
            g0 = 2 * t
            fire(g0 + 1, 1)
            drain_and_store(g0, 0)

            @pl.when(t < ngroups // 2 - 1)
            def _():
                fire(g0 + 2, 0)

            drain_and_store(g0 + 1, 1)
            return carry

        lax.fori_loop(0, ngroups // 2, body, 0)

    return gather_k


def _make_finish(blk, k_):
    e2 = blk * k_ // 2

    def finish_body(g_ref, pc_ref, ct_ref, mpt_ref, mpr_ref, ssel_ref,
                    w1b_ref, sd_ref, b1q_ref, w2a_ref, w2b_ref, o_ref):
        g = g_ref[...]
        c2 = jnp.dot(pc_ref[...], ct_ref[...],
                     preferred_element_type=jnp.float32)
        rep = jnp.broadcast_to(
            c2[:, None, :], (blk, k_ // 2, 128)).reshape(e2, 128)
        diffm = (rep - g) * mpt_ref[...]
        ssq = jnp.dot(diffm * diffm, ssel_ref[...],
                      preferred_element_type=jnp.float32)
        dist = jnp.sqrt(ssq + 1e-12)
        g1 = (
            jnp.dot(diffm, w1b_ref[...], preferred_element_type=jnp.float32)
            + jnp.dot(dist, sd_ref[...], preferred_element_type=jnp.float32)
            + b1q_ref[...]
        )
        g1 = jnp.where(g1 >= 0, g1, 0.2 * g1)
        za = jnp.dot(g1, w2a_ref[...], preferred_element_type=jnp.float32)
        zb = jnp.dot(g1, w2b_ref[...], preferred_element_type=jnp.float32)
        bc = g_ref.bitcast(jnp.bfloat16)[...]
        bc3 = bc.reshape(e2, 2, 128)
        gpe = bc3[:, 0, :].astype(jnp.float32) * mpr_ref[...]
        gpo = bc3[:, 1, :].astype(jnp.float32) * mpr_ref[...]
        se = za + gpe
        se = jnp.where(se >= 0, se, 0.2 * se)
        so = zb + gpo
        so = jnp.where(so >= 0, so, 0.2 * so)
        sse = jnp.sum(se.reshape(blk, k_ // 2, 128), axis=1)
        sso = jnp.sum(so.reshape(blk, k_ // 2, 128), axis=1)
        outa = sse[:, 0:32] + sse[:, 64:96]
        outb = sso[:, 0:32] + sso[:, 64:96]
        o_ref[...] = jnp.concatenate([outa, outb], axis=1) * (1.0 / k_)

    return finish_body


def kernel(points, features, knn_idx, W1, b1, W2, b2):
    b_, n_, _ = points.shape
    k_ = knn_idx.shape[1]
    d_ = features.shape[-1]
    dh = W2.shape[1]
    e_ = n_ * k_

    pts = points.reshape(n_, 3)
    feats = features.reshape(n_, d_)

    w2_top = W2[:d_]
    w2_bot = W2[d_:]
    b1r = b1.reshape(1, dh)
    b2r = b2.reshape(1, dh)
    eye16 = jnp.eye(16, dtype=jnp.float32)
    w1_pad = jnp.zeros((16, dh), jnp.float32).at[:3].set(W1[:3])
    w1d = W1[3]

    f32z = functools.partial(jnp.zeros, dtype=jnp.float32)
    lanes = jnp.arange(128)
    mpt = ((lanes % 64) >= 32).astype(jnp.float32).reshape(1, 128)
    mpr = ((lanes % 64) < 32).astype(jnp.float32).reshape(1, 128)
    ct = f32z((16, 128))
    ssel = f32z((128, 2))
    w1b = f32z((128, 128))
    sd = f32z((2, 128))
    w2a = f32z((128, 128))
    w2b = f32z((128, 128))
    for t in range(2):
        o = 64 * t
        ct = ct.at[:, o + 32:o + 48].set(eye16)
        ssel = ssel.at[o + 32:o + 64, t].set(1.0)
        w1b = w1b.at[o + 32:o + 48, o:o + 64].set(w1_pad)
        sd = sd.at[t, o:o + 64].set(w1d)
        w2a = w2a.at[o:o + 64, o:o + 32].set(w2_bot[:, 0:32])
        w2b = w2b.at[o:o + 64, o:o + 32].set(w2_bot[:, 32:64])
    b1q = jnp.concatenate([b1r, b1r], axis=1)

    cpt = (e_ + _NW * _CH - 1) // (_NW * _CH)
    cpt = ((cpt + 7) // 8) * 8
    ep = cpt * _NW * _CH
    idx_flat = jnp.pad(knn_idx.reshape(-1), (0, ep - e_))
    idx2d = idx_flat.reshape(ep // _CH, _CH)

    proj = pl.pallas_call(
        _proj_body,
        out_shape=jax.ShapeDtypeStruct((n_, dh), jnp.bfloat16),
    )(feats, w2_top, b2r)
    pack = lax.bitcast_convert_type(
        jnp.stack([proj[:, 0:32], proj[:, 32:64]], axis=-1),
        jnp.float32)
    pts_pad = f32z((n_, 32)).at[:, :3].set(pts)
    tbl = jnp.concatenate([pack, pts_pad], axis=1)

    graw = _make_gather(ep, cpt)(idx2d, tbl)
    gall = graw.reshape(ep // 2, 128)

    blk = 400
    nb = n_ // blk
    e2b = blk * k_ // 2
    full = lambda i: (0, 0)
    out = pl.pallas_call(
        _make_finish(blk, k_),
        grid=(nb,),
        in_specs=[
            pl.BlockSpec((e2b, 128), lambda i: (i, 0)),
            pl.BlockSpec((blk, 16), lambda i: (i, 0)),
            pl.BlockSpec((16, 128), full),
            pl.BlockSpec((1, 128), full),
            pl.BlockSpec((1, 128), full),
            pl.BlockSpec((128, 2), full),
            pl.BlockSpec((128, 128), full),
            pl.BlockSpec((2, 128), full),
            pl.BlockSpec((1, 128), full),
            pl.BlockSpec((128, 128), full),
            pl.BlockSpec((128, 128), full),
        ],
        out_specs=pl.BlockSpec((blk, dh), lambda i: (i, 0)),
        out_shape=jax.ShapeDtypeStruct((n_, dh), jnp.float32),
    )(gall, pts_pad[:, :16], ct, mpt, mpr, ssel, w1b, sd, b1q, w2a, w2b)

    return out.reshape(b_, n_, dh)


# speedup vs baseline: 1.3849x; 1.3033x over previous
import functools

import jax
import jax.numpy as jnp
from jax import lax
from jax.experimental import pallas as pl
from jax.experimental.pallas import tpu as pltpu
from jax.experimental.pallas import tpu_sc as plsc

_NC = 2
_NS = 16
_NW = _NC * _NS
_CH = 128
_WT = 64


def _proj_body(f_ref, w_ref, b_ref, o_ref):
    o_ref[...] = (
        jnp.dot(f_ref[...], w_ref[...], preferred_element_type=jnp.float32)
        + b_ref[...]
    ).astype(jnp.bfloat16)


def _make_gather(ep, cpt):
    mesh = plsc.VectorSubcoreMesh(
        core_axis_name="c", subcore_axis_name="s",
        num_cores=_NC, num_subcores=_NS,
    )

    nbuf = 4
    ngroups = cpt // nbuf

    @functools.partial(
        pl.kernel,
        out_type=jax.ShapeDtypeStruct((ep, _WT), jnp.float32),
        mesh=mesh,
        scratch_types=[
            pltpu.VMEM((cpt, _CH), jnp.int32),
            pltpu.VMEM((2, nbuf, _CH, _WT), jnp.float32),
            pltpu.SemaphoreType.DMA,
            pltpu.SemaphoreType.DMA,
        ],
        compiler_params=pltpu.CompilerParams(use_tc_tiling_on_sc=False),
    )
    def gather_k(idx_hbm, tbl_hbm, out_hbm, idx_v, buf, sem0, sem1):
        wid = lax.axis_index("s") * _NC + lax.axis_index("c")
        row0 = wid * cpt
        pltpu.sync_copy(idx_hbm.at[pl.ds(row0, cpt)], idx_v)
        sems = (sem0, sem1)

        def fire(g, h):
            for b in range(nbuf):
                i = g * nbuf + b
                pltpu.async_copy(tbl_hbm.at[idx_v.at[i]], buf.at[h, b],
                                 sems[h])

        def drain_and_store(g, h):
            for b in range(nbuf):
                pltpu.make_async_copy(
                    tbl_hbm.at[pl.ds(0, _CH)], buf.at[h, b], sems[h]).wait()
            for b in range(nbuf):
                c = row0 + (g * nbuf + b)
                pltpu.sync_copy(buf.at[h, b],
                                out_hbm.at[pl.ds(c * _CH, _CH)])

        fire(0, 0)

        def body(t, carry):
            g0 = 2 * t
            fire(g0 + 1, 1)
            drain_and_store(g0, 0)

            @pl.when(t < ngroups // 2 - 1)
            def _():
                fire(g0 + 2, 0)

            drain_and_store(g0 + 1, 1)
            return carry

        lax.fori_loop(0, ngroups // 2, body, 0)

    return gather_k


def _make_finish(blk, k_):
    e2 = blk * k_ // 2

    def finish_body(g_ref, pc_ref, ct_ref, mpt_ref, mpr_ref, ssel_ref,
                    w1b_ref, sd_ref, b1q_ref, w2a_ref, w2b_ref, o_ref):
        g = g_ref[...]
        c2 = jnp.dot(pc_ref[...], ct_ref[...],
                     preferred_element_type=jnp.float32)
        g3 = g.reshape(blk, k_ // 2, 128)
        diffm = ((c2[:, None, :] - g3)
                 * mpt_ref[...]).reshape(e2, 128)
        ssq = jnp.dot(diffm * diffm, ssel_ref[...],
                      preferred_element_type=jnp.float32)
        dist = jnp.sqrt(ssq + 1e-12)
        g1 = (
            jnp.dot(diffm, w1b_ref[...], preferred_element_type=jnp.float32)
            + jnp.dot(dist, sd_ref[...], preferred_element_type=jnp.float32)
            + b1q_ref[...]
        )
        g1 = jnp.where(g1 >= 0, g1, 0.2 * g1)
        za = jnp.dot(g1, w2a_ref[...], preferred_element_type=jnp.float32)
        zb = jnp.dot(g1, w2b_ref[...], preferred_element_type=jnp.float32)
        gi = g_ref.bitcast(jnp.int32)[...]
        gpe = lax.bitcast_convert_type(gi << 16, jnp.float32) * mpr_ref[...]
        gpo = lax.bitcast_convert_type(
            gi & jnp.int32(-65536), jnp.float32) * mpr_ref[...]
        se = za + gpe
        se = jnp.where(se >= 0, se, 0.2 * se)
        so = zb + gpo
        so = jnp.where(so >= 0, so, 0.2 * so)
        sse = jnp.sum(se.reshape(blk, k_ // 2, 128), axis=1)
        sso = jnp.sum(so.reshape(blk, k_ // 2, 128), axis=1)
        outa = sse[:, 0:32] + sse[:, 64:96]
        outb = sso[:, 0:32] + sso[:, 64:96]
        o_ref[...] = jnp.concatenate([outa, outb], axis=1) * (1.0 / k_)

    return finish_body


def kernel(points, features, knn_idx, W1, b1, W2, b2):
    b_, n_, _ = points.shape
    k_ = knn_idx.shape[1]
    d_ = features.shape[-1]
    dh = W2.shape[1]
    e_ = n_ * k_

    pts = points.reshape(n_, 3)
    feats = features.reshape(n_, d_)

    w2_top = W2[:d_]
    w2_bot = W2[d_:]
    b1r = b1.reshape(1, dh)
    b2r = b2.reshape(1, dh)
    eye16 = jnp.eye(16, dtype=jnp.float32)
    w1_pad = jnp.zeros((16, dh), jnp.float32).at[:3].set(W1[:3])
    w1d = W1[3]

    f32z = functools.partial(jnp.zeros, dtype=jnp.float32)
    lanes = jnp.arange(128)
    mpt = ((lanes % 64) >= 32).astype(jnp.float32).reshape(1, 128)
    mpr = ((lanes % 64) < 32).astype(jnp.float32).reshape(1, 128)
    ct = f32z((16, 128))
    ssel = f32z((128, 2))
    w1b = f32z((128, 128))
    sd = f32z((2, 128))
    w2a = f32z((128, 128))
    w2b = f32z((128, 128))
    for t in range(2):
        o = 64 * t
        ct = ct.at[:, o + 32:o + 48].set(eye16)
        ssel = ssel.at[o + 32:o + 64, t].set(1.0)
        w1b = w1b.at[o + 32:o + 48, o:o + 64].set(w1_pad)
        sd = sd.at[t, o:o + 64].set(w1d)
        w2a = w2a.at[o:o + 64, o:o + 32].set(w2_bot[:, 0:32])
        w2b = w2b.at[o:o + 64, o:o + 32].set(w2_bot[:, 32:64])
    b1q = jnp.concatenate([b1r, b1r], axis=1)

    cpt = (e_ + _NW * _CH - 1) // (_NW * _CH)
    cpt = ((cpt + 7) // 8) * 8
    ep = cpt * _NW * _CH
    idx_flat = jnp.pad(knn_idx.reshape(-1), (0, ep - e_))
    idx2d = idx_flat.reshape(ep // _CH, _CH)

    proj = pl.pallas_call(
        _proj_body,
        out_shape=jax.ShapeDtypeStruct((n_, dh), jnp.bfloat16),
    )(feats, w2_top, b2r)
    pack = lax.bitcast_convert_type(
        jnp.stack([proj[:, 0:32], proj[:, 32:64]], axis=-1),
        jnp.float32)
    pts_pad = f32z((n_, 32)).at[:, :3].set(pts)
    tbl = jnp.concatenate([pack, pts_pad], axis=1)

    graw = _make_gather(ep, cpt)(idx2d, tbl)
    gall = graw.reshape(ep // 2, 128)

    blk = 400
    nb = n_ // blk
    e2b = blk * k_ // 2
    full = lambda i: (0, 0)
    out = pl.pallas_call(
        _make_finish(blk, k_),
        grid=(nb,),
        in_specs=[
            pl.BlockSpec((e2b, 128), lambda i: (i, 0)),
            pl.BlockSpec((blk, 16), lambda i: (i, 0)),
            pl.BlockSpec((16, 128), full),
            pl.BlockSpec((1, 128), full),
            pl.BlockSpec((1, 128), full),
            pl.BlockSpec((128, 2), full),
            pl.BlockSpec((128, 128), full),
            pl.BlockSpec((2, 128), full),
            pl.BlockSpec((1, 128), full),
            pl.BlockSpec((128, 128), full),
            pl.BlockSpec((128, 128), full),
        ],
        out_specs=pl.BlockSpec((blk, dh), lambda i: (i, 0)),
        out_shape=jax.ShapeDtypeStruct((n_, dh), jnp.float32),
    )(gall, pts_pad[:, :16], ct, mpt, mpr, ssel, w1b, sd, b1q, w2a, w2b)

    return out.reshape(b_, n_, dh)
